# baseline (speedup 1.0000x reference)
import functools

import jax
import jax.numpy as jnp
from jax import lax
from jax.experimental import pallas as pl
from jax.experimental.pallas import tpu as pltpu
from jax.experimental.pallas import tpu_sc as plsc

_NUM_FEATURES = 26
_D = 16
_B = 16384
_NC = 2
_NS = 16
_NW = _NC * _NS
_BPW = _B // _NW


@functools.partial(
    pl.kernel,
    out_type=jax.ShapeDtypeStruct((_B, _NUM_FEATURES * _D), jnp.float32),
    mesh=plsc.VectorSubcoreMesh(core_axis_name="c", subcore_axis_name="s"),
    scratch_types=[
        pltpu.VMEM((_BPW,), jnp.int32),
        pltpu.VMEM((_BPW, _D), jnp.float32),
        pltpu.SemaphoreType.DMA,
    ],
    compiler_params=pltpu.CompilerParams(use_tc_tiling_on_sc=False),
)
def _cat_embed(*refs):
    ins = refs[:2 * _NUM_FEATURES]
    out = refs[2 * _NUM_FEATURES]
    idx_v, rows_v, sem = refs[2 * _NUM_FEATURES + 1:]

    wid = lax.axis_index("s") * _NC + lax.axis_index("c")
    base = wid * _BPW

    for j in range(_NUM_FEATURES):
        idx_hbm = ins[2 * j]
        tab_hbm = ins[2 * j + 1]
        pltpu.sync_copy(idx_hbm.at[pl.ds(base, _BPW)], idx_v)
        pltpu.async_copy(tab_hbm.at[idx_v], rows_v, sem).wait()
        pltpu.sync_copy(rows_v, out.at[pl.ds(base, _BPW), pl.ds(j * _D, _D)])


def kernel(f00, W_f00, f01, W_f01, f02, W_f02, f03, W_f03, f04, W_f04,
           f05, W_f05, f06, W_f06, f07, W_f07, f08, W_f08, f09, W_f09,
           f10, W_f10, f11, W_f11, f12, W_f12, f13, W_f13, f14, W_f14,
           f15, W_f15, f16, W_f16, f17, W_f17, f18, W_f18, f19, W_f19,
           f20, W_f20, f21, W_f21, f22, W_f22, f23, W_f23, f24, W_f24,
           f25, W_f25):
    args = (f00, W_f00, f01, W_f01, f02, W_f02, f03, W_f03, f04, W_f04,
            f05, W_f05, f06, W_f06, f07, W_f07, f08, W_f08, f09, W_f09,
            f10, W_f10, f11, W_f11, f12, W_f12, f13, W_f13, f14, W_f14,
            f15, W_f15, f16, W_f16, f17, W_f17, f18, W_f18, f19, W_f19,
            f20, W_f20, f21, W_f21, f22, W_f22, f23, W_f23, f24, W_f24,
            f25, W_f25)
    return _cat_embed(*args)


# speedup vs baseline: 1.0347x; 1.0347x over previous
import functools

import jax
import jax.numpy as jnp
from jax import lax
from jax.experimental import pallas as pl
from jax.experimental.pallas import tpu as pltpu
from jax.experimental.pallas import tpu_sc as plsc

_NUM_FEATURES = 26
_D = 16
_B = 16384
_NC = 2
_NS = 16
_NW = _NC * _NS
_BPW = _B // _NW


@functools.partial(
    pl.kernel,
    out_type=jax.ShapeDtypeStruct((_B, _NUM_FEATURES * _D), jnp.float32),
    mesh=plsc.VectorSubcoreMesh(core_axis_name="c", subcore_axis_name="s"),
    scratch_types=(
        [pltpu.VMEM((_NUM_FEATURES, _BPW), jnp.int32)]
        + [pltpu.VMEM((_BPW, _D), jnp.float32) for _ in range(8)]
        + [pltpu.SemaphoreType.DMA]
        + [pltpu.SemaphoreType.DMA for _ in range(8)]
        + [pltpu.SemaphoreType.DMA for _ in range(8)]
    ),
    compiler_params=pltpu.CompilerParams(use_tc_tiling_on_sc=False),
)
def _cat_embed(*refs):
    NBUF = 8
    G = 4

    ins = refs[:2 * _NUM_FEATURES]
    out = refs[2 * _NUM_FEATURES]
    rest = refs[2 * _NUM_FEATURES + 1:]
    idx_v = rest[0]
    bufs = rest[1:1 + NBUF]
    isem = rest[1 + NBUF]
    gsems = rest[2 + NBUF:2 + 2 * NBUF]
    wsems = rest[2 + 2 * NBUF:2 + 3 * NBUF]

    wid = lax.axis_index("s") * _NC + lax.axis_index("c")
    base = wid * _BPW

    idx_copies = [
        pltpu.async_copy(ins[2 * j].at[pl.ds(base, _BPW)], idx_v.at[j], isem)
        for j in range(_NUM_FEATURES)
    ]
    for c in idx_copies:
        c.wait()

    def fire_gather(j):
        b = j % NBUF
        return pltpu.async_copy(ins[2 * j + 1].at[idx_v.at[j]], bufs[b], gsems[b])

    def fire_write(j):
        b = j % NBUF
        return pltpu.async_copy(
            bufs[b], out.at[pl.ds(base, _BPW), pl.ds(j * _D, _D)], wsems[b])

    gathers = {}
    writes = {}
    for j in range(G):
        gathers[j] = fire_gather(j)

    for j in range(_NUM_FEATURES):
        b = j % NBUF
        gathers[j].wait()
        writes[j] = fire_write(j)
        n = j + G
        if n < _NUM_FEATURES:
            if n >= NBUF:
                writes[n - NBUF].wait()
            gathers[n] = fire_gather(n)

    for j in range(_NUM_FEATURES - NBUF, _NUM_FEATURES):
        writes[j].wait()


def kernel(f00, W_f00, f01, W_f01, f02, W_f02, f03, W_f03, f04, W_f04,
           f05, W_f05, f06, W_f06, f07, W_f07, f08, W_f08, f09, W_f09,
           f10, W_f10, f11, W_f11, f12, W_f12, f13, W_f13, f14, W_f14,
           f15, W_f15, f16, W_f16, f17, W_f17, f18, W_f18, f19, W_f19,
           f20, W_f20, f21, W_f21, f22, W_f22, f23, W_f23, f24, W_f24,
           f25, W_f25):
    args = (f00, W_f00, f01, W_f01, f02, W_f02, f03, W_f03, f04, W_f04,
            f05, W_f05, f06, W_f06, f07, W_f07, f08, W_f08, f09, W_f09,
            f10, W_f10, f11, W_f11, f12, W_f12, f13, W_f13, f14, W_f14,
            f15, W_f15, f16, W_f16, f17, W_f17, f18, W_f18, f19, W_f19,
            f20, W_f20, f21, W_f21, f22, W_f22, f23, W_f23, f24, W_f24,
            f25, W_f25)
    return _cat_embed(*args)
